# baseline (speedup 1.0000x reference)
import functools

import jax
import jax.numpy as jnp
from jax import lax
from jax.experimental import pallas as pl
from jax.experimental.pallas import tpu as pltpu
from jax.experimental.pallas import tpu_sc as plsc

G = 316
NN = G * G
HID = 64
C = 32768
T = 4
NPAD = 101120
RS = T * C
LEN_S = 32 * RS
Q = NPAD // 4
Q8 = NPAD // 8
K = 6320
H = 320
WNODES = K + 2 * H
MAIN = WNODES - H
ROWS = K // G


def _sc_dots(b0, x_ref, s_ref, w0_v, w1_v, s0_v, s1_v, s2_v, s3_v):
    wid = lax.axis_index("s") * 2 + lax.axis_index("c")
    bl = wid // 8
    b = b0 + bl
    q = wid % 8
    iota = lax.iota(jnp.int32, 16)
    sbufs = (s0_v, s1_v, s2_v, s3_v)
    wins = (w0_v, w1_v)

    for j in range(2):
        nb = q * Q8 + j * K
        for c in range(2):
            cb = (2 * b + c) * NN
            if j == 0:
                d0 = jnp.where(q == 0, NN - H, nb - H)
            else:
                d0 = nb - H
            pltpu.sync_copy(x_ref.at[pl.ds(cb + d0, H)],
                            wins[c].at[pl.ds(0, H)])
            if j == 1:
                m0 = jnp.where(q == 7, NN - MAIN, nb)
            else:
                m0 = nb
            pltpu.sync_copy(x_ref.at[pl.ds(cb + m0, MAIN)],
                            wins[c].at[pl.ds(H, MAIN)])
        if j == 1:
            @pl.when(q == 7)
            def _():
                nb3 = 7 * Q8 + K
                for c in range(2):
                    cb = (2 * b + c) * NN
                    pltpu.sync_copy(x_ref.at[pl.ds(cb + nb3, NN - nb3)],
                                    wins[c].at[pl.ds(H, NN - nb3)])
                    pltpu.sync_copy(x_ref.at[pl.ds(cb, H)],
                                    wins[c].at[pl.ds(H + NN - nb3, H)])

        def dots(i, _):
            m = H + 16 * i
            c0 = w0_v[pl.ds(m, 16)]
            c1 = w1_v[pl.ds(m, 16)]
            s0_v[pl.ds(16 * i, 16)] = (c0 * w0_v[pl.ds(m + 316, 16)] +
                                       c1 * w1_v[pl.ds(m + 316, 16)])
            s1_v[pl.ds(16 * i, 16)] = (c0 * w0_v[pl.ds(m - 316, 16)] +
                                       c1 * w1_v[pl.ds(m - 316, 16)])
            s2_v[pl.ds(16 * i, 16)] = (c0 * w0_v[pl.ds(m - 1, 16)] +
                                       c1 * w1_v[pl.ds(m - 1, 16)])
            s3_v[pl.ds(16 * i, 16)] = (c0 * w0_v[pl.ds(m + 1, 16)] +
                                       c1 * w1_v[pl.ds(m + 1, 16)])
            return 0
        lax.fori_loop(0, K // 16, dots, 0)

        for f in range(2):
            rr = 16 * f + iota
            msk = rr < ROWS
            rr = jnp.minimum(rr, ROWS - 1)
            mrow = H + rr * G
            cl0 = plsc.load_gather(w0_v, [mrow], mask=msk)
            cl1 = plsc.load_gather(w1_v, [mrow], mask=msk)
            nl0 = plsc.load_gather(w0_v, [mrow + 315], mask=msk)
            nl1 = plsc.load_gather(w1_v, [mrow + 315], mask=msk)
            plsc.store_scatter(s2_v, [rr * G], cl0 * nl0 + cl1 * nl1, mask=msk)
            plsc.store_scatter(s3_v, [rr * G + 315],
                               nl0 * cl0 + nl1 * cl1, mask=msk)

        for d in range(4):
            pltpu.sync_copy(sbufs[d],
                            s_ref.at[pl.ds((bl * 4 + d) * RS + nb, K)])


def _make_sc_stage(b0):
    @functools.partial(
        pl.kernel,
        mesh=plsc.VectorSubcoreMesh(core_axis_name="c", subcore_axis_name="s"),
        out_type=jax.ShapeDtypeStruct((16 * RS,), jnp.float32),
        compiler_params=pltpu.CompilerParams(needs_layout_passes=False),
        scratch_types=[
            pltpu.VMEM((WNODES,), jnp.float32),
            pltpu.VMEM((WNODES,), jnp.float32),
            pltpu.VMEM((K,), jnp.float32),
            pltpu.VMEM((K,), jnp.float32),
            pltpu.VMEM((K,), jnp.float32),
            pltpu.VMEM((K,), jnp.float32),
        ],
    )
    def _sc_stage(x_ref, s_ref, w0_v, w1_v, s0_v, s1_v, s2_v, s3_v):
        _sc_dots(b0, x_ref, s_ref, w0_v, w1_v, s0_v, s1_v, s2_v, s3_v)
    return _sc_stage


_sc_stage_lo = _make_sc_stage(0)
_sc_stage_hi = _make_sc_stage(4)


def _mlp_kernel(s0_ref, s1_ref, s2_ref, s3_ref, wembT_ref, bembT_ref,
                whidT_ref, bhidT_ref, wpost_ref, bpost_ref, out_ref, acc_ref):
    t = pl.program_id(1)

    @pl.when(t == 0)
    def _():
        acc_ref[...] = jnp.zeros_like(acc_ref)

    S = jnp.concatenate(
        [s0_ref[...].reshape(1, C), s1_ref[...].reshape(1, C),
         s2_ref[...].reshape(1, C), s3_ref[...].reshape(1, C)], axis=0)
    h1 = jnp.dot(wembT_ref[...].astype(jnp.bfloat16), S.astype(jnp.bfloat16),
                 preferred_element_type=jnp.float32)
    h1 = jnp.maximum(h1 + bembT_ref[...], 0.0)
    h2 = jnp.dot(whidT_ref[...].astype(jnp.bfloat16),
                 h1.astype(jnp.bfloat16), preferred_element_type=jnp.float32)
    h2 = jnp.maximum(h2 + bhidT_ref[...], 0.0)

    @pl.when(t < T - 1)
    def _():
        acc_ref[:, :1] += jnp.sum(h2, axis=1, keepdims=True)

    @pl.when(t == T - 1)
    def _():
        gidx = t * C + jax.lax.broadcasted_iota(jnp.int32, (1, C), 1)
        h2m = jnp.where(gidx < NN, h2, 0.0)
        acc = acc_ref[:, :1] + jnp.sum(h2m, axis=1, keepdims=True)
        res = jnp.sum(acc * wpost_ref[...]) + NN * bpost_ref[0, 0]
        out_ref[...] = jnp.full((1, 8, 128), res, dtype=jnp.float32)


@jax.jit
def kernel(x, W_emb, b_emb, W_hid, b_hid, W_post, b_post):
    B = x.shape[0]
    xpl = jnp.transpose(x, (0, 2, 1)).reshape(B * 2 * NN)
    s_lo = _sc_stage_lo(xpl)
    s_hi = _sc_stage_hi(xpl)

    mlp = lambda s_flat: pl.pallas_call(
        _mlp_kernel,
        grid=(4, T),
        in_specs=[
            pl.BlockSpec((C,), lambda b, t, d=d: ((4 * b + d) * T + t,))
            for d in range(4)
        ] + [
            pl.BlockSpec((HID, 4), lambda b, t: (0, 0)),
            pl.BlockSpec((HID, 1), lambda b, t: (0, 0)),
            pl.BlockSpec((HID, HID), lambda b, t: (0, 0)),
            pl.BlockSpec((HID, 1), lambda b, t: (0, 0)),
            pl.BlockSpec((HID, 1), lambda b, t: (0, 0)),
            pl.BlockSpec((1, 1), lambda b, t: (0, 0)),
        ],
        out_specs=pl.BlockSpec((1, 8, 128), lambda b, t: (b, 0, 0)),
        out_shape=jax.ShapeDtypeStruct((4, 8, 128), jnp.float32),
        scratch_shapes=[pltpu.VMEM((HID, 128), jnp.float32)],
    )(s_flat, s_flat, s_flat, s_flat, W_emb.T, b_emb[:, None],
      W_hid.T, b_hid[:, None], W_post, b_post.reshape(1, 1))
    out = jnp.concatenate([mlp(s_lo), mlp(s_hi)], axis=0)
    return out[:, 0, :1]


# speedup vs baseline: 1.0515x; 1.0515x over previous
import functools

import jax
import jax.numpy as jnp
from jax import lax
from jax.experimental import pallas as pl
from jax.experimental.pallas import tpu as pltpu
from jax.experimental.pallas import tpu_sc as plsc

G = 316
NN = G * G
HID = 64
C = 16384
T = 7
NPAD = 101120
RS = T * C
LEN_S = 32 * RS
Q = NPAD // 4
Q8 = NPAD // 8
K = 6320
H = 320
WNODES = K + 2 * H
MAIN = WNODES - H
ROWS = K // G


def _sc_dots(b0, x_ref, s_ref, w0_v, w1_v, s0_v, s1_v, s2_v, s3_v):
    wid = lax.axis_index("s") * 2 + lax.axis_index("c")
    bl = wid // 8
    b = b0 + bl
    q = wid % 8
    iota = lax.iota(jnp.int32, 16)
    sbufs = (s0_v, s1_v, s2_v, s3_v)
    wins = (w0_v, w1_v)

    for j in range(2):
        nb = q * Q8 + j * K
        for c in range(2):
            cb = (2 * b + c) * NN
            if j == 0:
                d0 = jnp.where(q == 0, NN - H, nb - H)
            else:
                d0 = nb - H
            pltpu.sync_copy(x_ref.at[pl.ds(cb + d0, H)],
                            wins[c].at[pl.ds(0, H)])
            if j == 1:
                m0 = jnp.where(q == 7, NN - MAIN, nb)
            else:
                m0 = nb
            pltpu.sync_copy(x_ref.at[pl.ds(cb + m0, MAIN)],
                            wins[c].at[pl.ds(H, MAIN)])
        if j == 1:
            @pl.when(q == 7)
            def _():
                nb3 = 7 * Q8 + K
                for c in range(2):
                    cb = (2 * b + c) * NN
                    pltpu.sync_copy(x_ref.at[pl.ds(cb + nb3, NN - nb3)],
                                    wins[c].at[pl.ds(H, NN - nb3)])
                    pltpu.sync_copy(x_ref.at[pl.ds(cb, H)],
                                    wins[c].at[pl.ds(H + NN - nb3, H)])

        def dots(i, _):
            m = H + 16 * i
            c0 = w0_v[pl.ds(m, 16)]
            c1 = w1_v[pl.ds(m, 16)]
            s0_v[pl.ds(16 * i, 16)] = (c0 * w0_v[pl.ds(m + 316, 16)] +
                                       c1 * w1_v[pl.ds(m + 316, 16)])
            s1_v[pl.ds(16 * i, 16)] = (c0 * w0_v[pl.ds(m - 316, 16)] +
                                       c1 * w1_v[pl.ds(m - 316, 16)])
            s2_v[pl.ds(16 * i, 16)] = (c0 * w0_v[pl.ds(m - 1, 16)] +
                                       c1 * w1_v[pl.ds(m - 1, 16)])
            s3_v[pl.ds(16 * i, 16)] = (c0 * w0_v[pl.ds(m + 1, 16)] +
                                       c1 * w1_v[pl.ds(m + 1, 16)])
            return 0
        lax.fori_loop(0, K // 16, dots, 0)

        for f in range(2):
            rr = 16 * f + iota
            msk = rr < ROWS
            rr = jnp.minimum(rr, ROWS - 1)
            mrow = H + rr * G
            cl0 = plsc.load_gather(w0_v, [mrow], mask=msk)
            cl1 = plsc.load_gather(w1_v, [mrow], mask=msk)
            nl0 = plsc.load_gather(w0_v, [mrow + 315], mask=msk)
            nl1 = plsc.load_gather(w1_v, [mrow + 315], mask=msk)
            plsc.store_scatter(s2_v, [rr * G], cl0 * nl0 + cl1 * nl1, mask=msk)
            plsc.store_scatter(s3_v, [rr * G + 315],
                               nl0 * cl0 + nl1 * cl1, mask=msk)

        for d in range(4):
            pltpu.sync_copy(sbufs[d],
                            s_ref.at[pl.ds((bl * 4 + d) * RS + nb, K)])


def _make_sc_stage(b0):
    @functools.partial(
        pl.kernel,
        mesh=plsc.VectorSubcoreMesh(core_axis_name="c", subcore_axis_name="s"),
        out_type=jax.ShapeDtypeStruct((16 * RS,), jnp.float32),
        compiler_params=pltpu.CompilerParams(needs_layout_passes=False),
        scratch_types=[
            pltpu.VMEM((WNODES,), jnp.float32),
            pltpu.VMEM((WNODES,), jnp.float32),
            pltpu.VMEM((K,), jnp.float32),
            pltpu.VMEM((K,), jnp.float32),
            pltpu.VMEM((K,), jnp.float32),
            pltpu.VMEM((K,), jnp.float32),
        ],
    )
    def _sc_stage(x_ref, s_ref, w0_v, w1_v, s0_v, s1_v, s2_v, s3_v):
        _sc_dots(b0, x_ref, s_ref, w0_v, w1_v, s0_v, s1_v, s2_v, s3_v)
    return _sc_stage


_sc_stage_lo = _make_sc_stage(0)
_sc_stage_hi = _make_sc_stage(4)


def _mlp_kernel(s0_ref, s1_ref, s2_ref, s3_ref, wembT_ref, bembT_ref,
                whidT_ref, bhidT_ref, wpost_ref, bpost_ref, out_ref, acc_ref):
    t = pl.program_id(1)

    @pl.when(t == 0)
    def _():
        acc_ref[...] = jnp.zeros_like(acc_ref)

    S = jnp.concatenate(
        [s0_ref[...].reshape(1, C), s1_ref[...].reshape(1, C),
         s2_ref[...].reshape(1, C), s3_ref[...].reshape(1, C)], axis=0)
    h1 = jnp.dot(wembT_ref[...].astype(jnp.bfloat16), S.astype(jnp.bfloat16),
                 preferred_element_type=jnp.float32)
    h1 = jnp.maximum(h1 + bembT_ref[...], 0.0)
    h2 = jnp.dot(whidT_ref[...].astype(jnp.bfloat16),
                 h1.astype(jnp.bfloat16), preferred_element_type=jnp.float32)
    h2 = jnp.maximum(h2 + bhidT_ref[...], 0.0)

    @pl.when(t < T - 1)
    def _():
        acc_ref[:, :1] += jnp.sum(h2, axis=1, keepdims=True)

    @pl.when(t == T - 1)
    def _():
        gidx = t * C + jax.lax.broadcasted_iota(jnp.int32, (1, C), 1)
        h2m = jnp.where(gidx < NN, h2, 0.0)
        acc = acc_ref[:, :1] + jnp.sum(h2m, axis=1, keepdims=True)
        res = jnp.sum(acc * wpost_ref[...]) + NN * bpost_ref[0, 0]
        out_ref[...] = jnp.full((1, 8, 128), res, dtype=jnp.float32)


@jax.jit
def kernel(x, W_emb, b_emb, W_hid, b_hid, W_post, b_post):
    B = x.shape[0]
    xpl = jnp.transpose(x, (0, 2, 1)).reshape(B * 2 * NN)
    s_lo = _sc_stage_lo(xpl)
    s_hi = _sc_stage_hi(xpl)

    mlp = lambda s_flat: pl.pallas_call(
        _mlp_kernel,
        grid=(4, T),
        in_specs=[
            pl.BlockSpec((C,), lambda b, t, d=d: ((4 * b + d) * T + t,))
            for d in range(4)
        ] + [
            pl.BlockSpec((HID, 4), lambda b, t: (0, 0)),
            pl.BlockSpec((HID, 1), lambda b, t: (0, 0)),
            pl.BlockSpec((HID, HID), lambda b, t: (0, 0)),
            pl.BlockSpec((HID, 1), lambda b, t: (0, 0)),
            pl.BlockSpec((HID, 1), lambda b, t: (0, 0)),
            pl.BlockSpec((1, 1), lambda b, t: (0, 0)),
        ],
        out_specs=pl.BlockSpec((1, 8, 128), lambda b, t: (b, 0, 0)),
        out_shape=jax.ShapeDtypeStruct((4, 8, 128), jnp.float32),
        scratch_shapes=[pltpu.VMEM((HID, 128), jnp.float32)],
    )(s_flat, s_flat, s_flat, s_flat, W_emb.T, b_emb[:, None],
      W_hid.T, b_hid[:, None], W_post, b_post.reshape(1, 1))
    out = jnp.concatenate([mlp(s_lo), mlp(s_hi)], axis=0)
    return out[:, 0, :1]
